# baseline (speedup 1.0000x reference)
import functools

import jax
import jax.numpy as jnp
from jax import lax
from jax.experimental import pallas as pl
from jax.experimental.pallas import tpu as pltpu
from jax.experimental.pallas import tpu_sc as plsc

_TOPK = 2
_TILE = 256
_L = 16
_NW = 32


def _plan_body(rl_ref, d0_ref, d1_ref, w0_ref, w1_ref, te_ref):
    logits = rl_ref[...]
    T, E = logits.shape
    NT = te_ref.shape[0]
    col = lax.broadcasted_iota(jnp.int32, (T, E), 1)
    m1 = jnp.max(logits, axis=1, keepdims=True)
    a1 = jnp.min(jnp.where(logits == m1, col, E), axis=1, keepdims=True)
    masked = jnp.where(col == a1, -jnp.inf, logits)
    m2 = jnp.max(masked, axis=1, keepdims=True)
    a2 = jnp.min(jnp.where(masked == m2, col, E), axis=1, keepdims=True)
    z = jnp.exp(m2 - m1)
    w0_ref[...] = 1.0 / (1.0 + z)
    w1_ref[...] = z / (1.0 + z)

    sel0 = col == a1
    sel1 = col == a2
    M = sel0.astype(jnp.int32) + sel1.astype(jnp.int32)
    x = M
    sh = 1
    while sh < T:
        x = jnp.concatenate(
            [jnp.zeros((sh, E), jnp.int32), x[:-sh, :]], axis=0) + x
        sh *= 2
    excl = x - M
    cnt = x[T - 1:T, :]
    padded = ((cnt + (_TILE - 1)) // _TILE) * _TILE
    r8 = lax.broadcasted_iota(jnp.int32, (E, E), 0)
    c8 = lax.broadcasted_iota(jnp.int32, (E, E), 1)
    U = (r8 < c8).astype(jnp.float32)
    base = jnp.dot(padded.astype(jnp.float32), U,
                   preferred_element_type=jnp.float32).astype(jnp.int32)
    destM = jnp.broadcast_to(base, (T, E)) + excl
    d0_ref[...] = jnp.sum(jnp.where(sel0, destM, 0), axis=1, keepdims=True)
    d1_ref[...] = jnp.sum(jnp.where(sel1, destM, 0), axis=1, keepdims=True)

    jt = lax.broadcasted_iota(jnp.int32, (NT, E), 0)
    endB = jnp.broadcast_to(base + padded, (NT, E))
    s = jnp.sum((jt * _TILE >= endB).astype(jnp.int32), axis=1, keepdims=True)
    te_ref[...] = jnp.minimum(s, E - 1)


def _gemm_body(te_ref, xh_ref, gw_ref, uw_ref, dw_ref, w_ref, yw_ref):
    xh = xh_ref[...].astype(jnp.bfloat16)
    g = jnp.dot(xh, gw_ref[0], preferred_element_type=jnp.float32)
    u = jnp.dot(xh, uw_ref[0], preferred_element_type=jnp.float32)
    act = (g * jax.nn.sigmoid(g)) * u
    y = jnp.dot(act.astype(jnp.bfloat16), dw_ref[0],
                preferred_element_type=jnp.float32)
    yw_ref[...] = y * w_ref[...]


def _make_gather(T, H, NP):
    rows_w = NP // _NW
    chunk = rows_w // 2
    n_scan = T // _L
    mesh = plsc.VectorSubcoreMesh(core_axis_name="c", subcore_axis_name="s")

    @functools.partial(
        pl.kernel,
        out_type=[jax.ShapeDtypeStruct((NP, H), jnp.float32),
                  jax.ShapeDtypeStruct((NP,), jnp.float32)],
        mesh=mesh,
        scratch_types=[
            pltpu.VMEM((T,), jnp.int32),
            pltpu.VMEM((T,), jnp.int32),
            pltpu.VMEM((T,), jnp.float32),
            pltpu.VMEM((T,), jnp.float32),
            pltpu.VMEM((rows_w,), jnp.int32),
            pltpu.VMEM((rows_w,), jnp.float32),
            pltpu.VMEM((chunk, H), jnp.float32),
            pltpu.SemaphoreType.DMA,
        ],
        compiler_params=pltpu.CompilerParams(needs_layout_passes=False),
    )
    def gather_k(d0_hbm, d1_hbm, w0_hbm, w1_hbm, hs_hbm,
                 xh_hbm, wrow_hbm,
                 d0_v, d1_v, w0_v, w1_v, src_v, wr_v, rows_v, sem):
        wid = lax.axis_index("s") * 2 + lax.axis_index("c")
        rbase = wid * rows_w

        pltpu.sync_copy(d0_hbm, d0_v)
        pltpu.sync_copy(d1_hbm, d1_v)
        pltpu.sync_copy(w0_hbm, w0_v)
        pltpu.sync_copy(w1_hbm, w1_v)

        zi = jnp.zeros((_L,), jnp.int32)
        zf = jnp.zeros((_L,), jnp.float32)
        def init_body(j, _):
            src_v[pl.ds(j * _L, _L)] = zi
            wr_v[pl.ds(j * _L, _L)] = zf
            return 0
        lax.fori_loop(0, rows_w // _L, init_body, 0)

        lanes = lax.broadcasted_iota(jnp.int32, (_L,), 0)

        def scan_body(j, _):
            toks = j * _L + lanes
            for dv, wv in ((d0_v, w0_v), (d1_v, w1_v)):
                idx = dv[pl.ds(j * _L, _L)] - rbase
                m = (idx >= 0) & (idx < rows_w)
                plsc.store_scatter(src_v, [idx], toks, mask=m)
                plsc.store_scatter(wr_v, [idx], wv[pl.ds(j * _L, _L)], mask=m)
            return 0
        lax.fori_loop(0, n_scan, scan_body, 0)

        pltpu.sync_copy(wr_v, wrow_hbm.at[pl.ds(rbase, rows_w)])
        for c in range(2):
            idx_ref = src_v.at[pl.ds(c * chunk, chunk)]
            pltpu.async_copy(hs_hbm.at[idx_ref], rows_v, sem).wait()
            pltpu.sync_copy(rows_v,
                            xh_hbm.at[pl.ds(rbase + c * chunk, chunk)])

    return gather_k


def _make_combine(T, H, NP):
    tok_w = T // _NW
    chunk = tok_w // 2
    ncol = H // _L
    mesh = plsc.VectorSubcoreMesh(core_axis_name="c", subcore_axis_name="s")

    @functools.partial(
        pl.kernel,
        out_type=jax.ShapeDtypeStruct((T, H), jnp.float32),
        mesh=mesh,
        scratch_types=[
            pltpu.VMEM((tok_w,), jnp.int32),
            pltpu.VMEM((tok_w,), jnp.int32),
            pltpu.VMEM((chunk, H), jnp.float32),
            pltpu.VMEM((chunk, H), jnp.float32),
            pltpu.SemaphoreType.DMA,
        ],
        compiler_params=pltpu.CompilerParams(needs_layout_passes=False),
    )
    def combine_k(d0_hbm, d1_hbm, yw_hbm, out_hbm,
                  i0_v, i1_v, a_v, b_v, sem):
        wid = lax.axis_index("s") * 2 + lax.axis_index("c")
        tbase = wid * tok_w
        pltpu.sync_copy(d0_hbm.at[pl.ds(tbase, tok_w)], i0_v)
        pltpu.sync_copy(d1_hbm.at[pl.ds(tbase, tok_w)], i1_v)

        for c in range(2):
            pltpu.async_copy(
                yw_hbm.at[i0_v.at[pl.ds(c * chunk, chunk)]], a_v, sem).wait()
            pltpu.async_copy(
                yw_hbm.at[i1_v.at[pl.ds(c * chunk, chunk)]], b_v, sem).wait()

            def add_body(r, _):
                for cc in range(ncol):
                    s = pl.ds(cc * _L, _L)
                    a_v[r, s] = a_v[r, s] + b_v[r, s]
                return 0
            lax.fori_loop(0, chunk, add_body, 0)
            pltpu.sync_copy(a_v, out_hbm.at[pl.ds(tbase + c * chunk, chunk)])

    return combine_k


@jax.jit
def kernel(hidden_states, router_logits, up_weight, down_weight):
    T, H = hidden_states.shape
    E = up_weight.shape[0]
    I = down_weight.shape[1]
    NT = (T * _TOPK) // _TILE + E
    NP = NT * _TILE

    d0, d1, w0, w1, te = pl.pallas_call(
        _plan_body,
        out_shape=[
            jax.ShapeDtypeStruct((T, 1), jnp.int32),
            jax.ShapeDtypeStruct((T, 1), jnp.int32),
            jax.ShapeDtypeStruct((T, 1), jnp.float32),
            jax.ShapeDtypeStruct((T, 1), jnp.float32),
            jax.ShapeDtypeStruct((NT, 1), jnp.int32),
        ],
    )(router_logits)
    d0 = d0.reshape(T)
    d1 = d1.reshape(T)
    te = te.reshape(NT)

    xh, wrow = _make_gather(T, H, NP)(
        d0, d1, w0.reshape(T), w1.reshape(T), hidden_states)

    gate_w = up_weight[:, :, :I].astype(jnp.bfloat16)
    up_w = up_weight[:, :, I:].astype(jnp.bfloat16)
    dw = down_weight.astype(jnp.bfloat16)

    yw = pl.pallas_call(
        _gemm_body,
        grid_spec=pltpu.PrefetchScalarGridSpec(
            num_scalar_prefetch=1,
            grid=(NT,),
            in_specs=[
                pl.BlockSpec((_TILE, H), lambda t, te: (t, 0)),
                pl.BlockSpec((1, H, I), lambda t, te: (te[t], 0, 0)),
                pl.BlockSpec((1, H, I), lambda t, te: (te[t], 0, 0)),
                pl.BlockSpec((1, I, H), lambda t, te: (te[t], 0, 0)),
                pl.BlockSpec((_TILE, 1), lambda t, te: (t, 0)),
            ],
            out_specs=pl.BlockSpec((_TILE, H), lambda t, te: (t, 0)),
        ),
        out_shape=jax.ShapeDtypeStruct((NP, H), jnp.float32),
        compiler_params=pltpu.CompilerParams(
            dimension_semantics=("arbitrary",),
        ),
    )(te, xh, gate_w, up_w, dw, wrow.reshape(NP, 1))

    return _make_combine(T, H, NP)(d0, d1, yw)


# speedup vs baseline: 1.4079x; 1.4079x over previous
import functools

import jax
import jax.numpy as jnp
from jax import lax
from jax.experimental import pallas as pl
from jax.experimental.pallas import tpu as pltpu
from jax.experimental.pallas import tpu_sc as plsc

_TOPK = 2
_TILE = 256
_L = 16
_NW = 32


def _plan_body(rl_ref, d0_ref, d1_ref, w0_ref, w1_ref, te_ref):
    logits = rl_ref[...]
    T, E = logits.shape
    NT = te_ref.shape[0]
    col = lax.broadcasted_iota(jnp.int32, (T, E), 1)
    m1 = jnp.max(logits, axis=1, keepdims=True)
    a1 = jnp.min(jnp.where(logits == m1, col, E), axis=1, keepdims=True)
    masked = jnp.where(col == a1, -jnp.inf, logits)
    m2 = jnp.max(masked, axis=1, keepdims=True)
    a2 = jnp.min(jnp.where(masked == m2, col, E), axis=1, keepdims=True)
    z = jnp.exp(m2 - m1)
    w0_ref[...] = 1.0 / (1.0 + z)
    w1_ref[...] = z / (1.0 + z)

    sel0 = col == a1
    sel1 = col == a2
    M = sel0.astype(jnp.int32) + sel1.astype(jnp.int32)
    x = M
    sh = 1
    while sh < T:
        x = jnp.concatenate(
            [jnp.zeros((sh, E), jnp.int32), x[:-sh, :]], axis=0) + x
        sh *= 2
    excl = x - M
    cnt = x[T - 1:T, :]
    padded = ((cnt + (_TILE - 1)) // _TILE) * _TILE
    r8 = lax.broadcasted_iota(jnp.int32, (E, E), 0)
    c8 = lax.broadcasted_iota(jnp.int32, (E, E), 1)
    U = (r8 < c8).astype(jnp.float32)
    base = jnp.dot(padded.astype(jnp.float32), U,
                   preferred_element_type=jnp.float32).astype(jnp.int32)
    destM = jnp.broadcast_to(base, (T, E)) + excl
    d0_ref[...] = jnp.sum(jnp.where(sel0, destM, 0), axis=1, keepdims=True)
    d1_ref[...] = jnp.sum(jnp.where(sel1, destM, 0), axis=1, keepdims=True)

    jt = lax.broadcasted_iota(jnp.int32, (NT, E), 0)
    endB = jnp.broadcast_to(base + padded, (NT, E))
    s = jnp.sum((jt * _TILE >= endB).astype(jnp.int32), axis=1, keepdims=True)
    te_ref[...] = jnp.minimum(s, E - 1)


def _gemm_body(te_ref, xh_ref, gw_ref, uw_ref, dw_ref, w_ref, yw_ref):
    xh = xh_ref[...].astype(jnp.bfloat16)
    g = jnp.dot(xh, gw_ref[0], preferred_element_type=jnp.float32)
    u = jnp.dot(xh, uw_ref[0], preferred_element_type=jnp.float32)
    act = (g * jax.nn.sigmoid(g)) * u
    y = jnp.dot(act.astype(jnp.bfloat16), dw_ref[0],
                preferred_element_type=jnp.float32)
    yw_ref[...] = y * w_ref[...]


def _make_gather(T, H, NP):
    rows_w = NP // _NW
    nchunk = 3
    chunk = rows_w // nchunk
    n_scan = T // _L
    mesh = plsc.VectorSubcoreMesh(core_axis_name="c", subcore_axis_name="s")

    @functools.partial(
        pl.kernel,
        out_type=[jax.ShapeDtypeStruct((NP, H), jnp.float32),
                  jax.ShapeDtypeStruct((NP,), jnp.float32)],
        mesh=mesh,
        scratch_types=[
            pltpu.VMEM((T,), jnp.int32),
            pltpu.VMEM((T,), jnp.int32),
            pltpu.VMEM((T,), jnp.float32),
            pltpu.VMEM((T,), jnp.float32),
            pltpu.VMEM((rows_w,), jnp.int32),
            pltpu.VMEM((rows_w,), jnp.float32),
            pltpu.VMEM((chunk, H), jnp.float32),
            pltpu.VMEM((chunk, H), jnp.float32),
            pltpu.SemaphoreType.DMA,
            pltpu.SemaphoreType.DMA,
        ],
        compiler_params=pltpu.CompilerParams(needs_layout_passes=False),
    )
    def gather_k(d0_hbm, d1_hbm, w0_hbm, w1_hbm, hs_hbm,
                 xh_hbm, wrow_hbm,
                 d0_v, d1_v, w0_v, w1_v, src_v, wr_v, rows_a, rows_b,
                 sem_a, sem_b):
        wid = lax.axis_index("s") * 2 + lax.axis_index("c")
        rbase = wid * rows_w

        pltpu.sync_copy(d0_hbm, d0_v)
        pltpu.sync_copy(d1_hbm, d1_v)
        pltpu.sync_copy(w0_hbm, w0_v)
        pltpu.sync_copy(w1_hbm, w1_v)

        lanes = lax.broadcasted_iota(jnp.int32, (_L,), 0)

        zf = jnp.zeros((_L,), jnp.float32)
        def init_body(j, _):
            src_v[pl.ds(j * _L, _L)] = lax.rem(rbase + j * _L + lanes, T)
            wr_v[pl.ds(j * _L, _L)] = zf
            return 0
        lax.fori_loop(0, rows_w // _L, init_body, 0)

        def scan_body(j, _):
            toks = j * _L + lanes
            for dv, wv in ((d0_v, w0_v), (d1_v, w1_v)):
                idx = dv[pl.ds(j * _L, _L)] - rbase
                m = (idx >= 0) & (idx < rows_w)
                plsc.store_scatter(src_v, [idx], toks, mask=m)
                plsc.store_scatter(wr_v, [idx], wv[pl.ds(j * _L, _L)], mask=m)
            return 0
        lax.fori_loop(0, n_scan, scan_body, 0)

        pltpu.sync_copy(wr_v, wrow_hbm.at[pl.ds(rbase, rows_w)])
        bufs = (rows_a, rows_b)
        sems = (sem_a, sem_b)
        handles = [None] * nchunk
        handles[0] = pltpu.async_copy(
            hs_hbm.at[src_v.at[pl.ds(0, chunk)]], bufs[0], sems[0])
        for c in range(nchunk):
            if c + 1 < nchunk:
                handles[c + 1] = pltpu.async_copy(
                    hs_hbm.at[src_v.at[pl.ds((c + 1) * chunk, chunk)]],
                    bufs[(c + 1) % 2], sems[(c + 1) % 2])
            handles[c].wait()
            pltpu.sync_copy(bufs[c % 2],
                            xh_hbm.at[pl.ds(rbase + c * chunk, chunk)])

    return gather_k


def _make_combine(T, H, NP):
    tok_w = T // _NW
    chunk = tok_w // 2
    ncol = H // _L
    mesh = plsc.VectorSubcoreMesh(core_axis_name="c", subcore_axis_name="s")

    @functools.partial(
        pl.kernel,
        out_type=jax.ShapeDtypeStruct((T, H), jnp.float32),
        mesh=mesh,
        scratch_types=[
            pltpu.VMEM((tok_w,), jnp.int32),
            pltpu.VMEM((tok_w,), jnp.int32),
            pltpu.VMEM((chunk, H), jnp.float32),
            pltpu.VMEM((chunk, H), jnp.float32),
            pltpu.SemaphoreType.DMA,
        ],
        compiler_params=pltpu.CompilerParams(needs_layout_passes=False),
    )
    def combine_k(d0_hbm, d1_hbm, yw_hbm, out_hbm,
                  i0_v, i1_v, a_v, b_v, sem):
        wid = lax.axis_index("s") * 2 + lax.axis_index("c")
        tbase = wid * tok_w
        pltpu.sync_copy(d0_hbm.at[pl.ds(tbase, tok_w)], i0_v)
        pltpu.sync_copy(d1_hbm.at[pl.ds(tbase, tok_w)], i1_v)

        for c in range(2):
            pltpu.async_copy(
                yw_hbm.at[i0_v.at[pl.ds(c * chunk, chunk)]], a_v, sem).wait()
            pltpu.async_copy(
                yw_hbm.at[i1_v.at[pl.ds(c * chunk, chunk)]], b_v, sem).wait()

            def add_body(r, _):
                for cc in range(ncol):
                    s = pl.ds(cc * _L, _L)
                    a_v[r, s] = a_v[r, s] + b_v[r, s]
                return 0
            lax.fori_loop(0, chunk, add_body, 0)
            pltpu.sync_copy(a_v, out_hbm.at[pl.ds(tbase + c * chunk, chunk)])

    return combine_k


@jax.jit
def kernel(hidden_states, router_logits, up_weight, down_weight):
    T, H = hidden_states.shape
    E = up_weight.shape[0]
    I = down_weight.shape[1]
    NT = (T * _TOPK) // _TILE + E
    NP = NT * _TILE

    d0, d1, w0, w1, te = pl.pallas_call(
        _plan_body,
        out_shape=[
            jax.ShapeDtypeStruct((T, 1), jnp.int32),
            jax.ShapeDtypeStruct((T, 1), jnp.int32),
            jax.ShapeDtypeStruct((T, 1), jnp.float32),
            jax.ShapeDtypeStruct((T, 1), jnp.float32),
            jax.ShapeDtypeStruct((NT, 1), jnp.int32),
        ],
    )(router_logits)
    d0 = d0.reshape(T)
    d1 = d1.reshape(T)
    te = te.reshape(NT)

    xh, wrow = _make_gather(T, H, NP)(
        d0, d1, w0.reshape(T), w1.reshape(T), hidden_states)

    gate_w = up_weight[:, :, :I].astype(jnp.bfloat16)
    up_w = up_weight[:, :, I:].astype(jnp.bfloat16)
    dw = down_weight.astype(jnp.bfloat16)

    yw = pl.pallas_call(
        _gemm_body,
        grid_spec=pltpu.PrefetchScalarGridSpec(
            num_scalar_prefetch=1,
            grid=(NT,),
            in_specs=[
                pl.BlockSpec((_TILE, H), lambda t, te: (t, 0)),
                pl.BlockSpec((1, H, I), lambda t, te: (te[t], 0, 0)),
                pl.BlockSpec((1, H, I), lambda t, te: (te[t], 0, 0)),
                pl.BlockSpec((1, I, H), lambda t, te: (te[t], 0, 0)),
                pl.BlockSpec((_TILE, 1), lambda t, te: (t, 0)),
            ],
            out_specs=pl.BlockSpec((_TILE, H), lambda t, te: (t, 0)),
        ),
        out_shape=jax.ShapeDtypeStruct((NP, H), jnp.float32),
        compiler_params=pltpu.CompilerParams(
            dimension_semantics=("arbitrary",),
        ),
    )(te, xh, gate_w, up_w, dw, wrow.reshape(NP, 1))

    return _make_combine(T, H, NP)(d0, d1, yw)


# speedup vs baseline: 1.9709x; 1.3999x over previous
import functools

import jax
import jax.numpy as jnp
from jax import lax
from jax.experimental import pallas as pl
from jax.experimental.pallas import tpu as pltpu
from jax.experimental.pallas import tpu_sc as plsc

_TOPK = 2
_TILE = 256
_L = 16
_NW = 32


def _plan_body(rl_ref, d0_ref, d1_ref, w0_ref, w1_ref, te_ref):
    logits = rl_ref[...]
    T, E = logits.shape
    NT = te_ref.shape[0]
    col = lax.broadcasted_iota(jnp.int32, (T, E), 1)
    m1 = jnp.max(logits, axis=1, keepdims=True)
    a1 = jnp.min(jnp.where(logits == m1, col, E), axis=1, keepdims=True)
    masked = jnp.where(col == a1, -jnp.inf, logits)
    m2 = jnp.max(masked, axis=1, keepdims=True)
    a2 = jnp.min(jnp.where(masked == m2, col, E), axis=1, keepdims=True)
    z = jnp.exp(m2 - m1)
    w0_ref[...] = 1.0 / (1.0 + z)
    w1_ref[...] = z / (1.0 + z)

    sel0 = col == a1
    sel1 = col == a2
    M = sel0.astype(jnp.int32) + sel1.astype(jnp.int32)
    x = M
    sh = 1
    while sh < T:
        x = jnp.concatenate(
            [jnp.zeros((sh, E), jnp.int32), x[:-sh, :]], axis=0) + x
        sh *= 2
    excl = x - M
    cnt = x[T - 1:T, :]
    padded = ((cnt + (_TILE - 1)) // _TILE) * _TILE
    r8 = lax.broadcasted_iota(jnp.int32, (E, E), 0)
    c8 = lax.broadcasted_iota(jnp.int32, (E, E), 1)
    U = (r8 < c8).astype(jnp.float32)
    base = jnp.dot(padded.astype(jnp.float32), U,
                   preferred_element_type=jnp.float32).astype(jnp.int32)
    destM = jnp.broadcast_to(base, (T, E)) + excl
    d0_ref[...] = jnp.sum(jnp.where(sel0, destM, 0), axis=1, keepdims=True)
    d1_ref[...] = jnp.sum(jnp.where(sel1, destM, 0), axis=1, keepdims=True)

    jt = lax.broadcasted_iota(jnp.int32, (NT, E), 0)
    endB = jnp.broadcast_to(base + padded, (NT, E))
    s = jnp.sum((jt * _TILE >= endB).astype(jnp.int32), axis=1, keepdims=True)
    te_ref[...] = jnp.minimum(s, E - 1)


def _gemm_body(te_ref, xh_ref, gw_ref, uw_ref, dw_ref, w_ref, yw_ref):
    xh = xh_ref[...].astype(jnp.bfloat16)
    g = jnp.dot(xh, gw_ref[0].astype(jnp.bfloat16),
                preferred_element_type=jnp.float32)
    u = jnp.dot(xh, uw_ref[0].astype(jnp.bfloat16),
                preferred_element_type=jnp.float32)
    act = (g * jax.nn.sigmoid(g)) * u
    y = jnp.dot(act.astype(jnp.bfloat16), dw_ref[0].astype(jnp.bfloat16),
                preferred_element_type=jnp.float32)
    yw_ref[...] = y * w_ref[...]


def _make_gather(T, H, NP):
    rows_w = NP // _NW
    nchunk = 3
    chunk = rows_w // nchunk
    n_scan = T // _L
    mesh = plsc.VectorSubcoreMesh(core_axis_name="c", subcore_axis_name="s")

    @functools.partial(
        pl.kernel,
        out_type=[jax.ShapeDtypeStruct((NP, H), jnp.float32),
                  jax.ShapeDtypeStruct((NP,), jnp.float32)],
        mesh=mesh,
        scratch_types=[
            pltpu.VMEM((T,), jnp.int32),
            pltpu.VMEM((T,), jnp.int32),
            pltpu.VMEM((T,), jnp.float32),
            pltpu.VMEM((T,), jnp.float32),
            pltpu.VMEM((rows_w,), jnp.int32),
            pltpu.VMEM((rows_w,), jnp.float32),
            pltpu.VMEM((chunk, H), jnp.float32),
            pltpu.VMEM((chunk, H), jnp.float32),
            pltpu.SemaphoreType.DMA,
            pltpu.SemaphoreType.DMA,
        ],
        compiler_params=pltpu.CompilerParams(needs_layout_passes=False),
    )
    def gather_k(d0_hbm, d1_hbm, w0_hbm, w1_hbm, hs_hbm,
                 xh_hbm, wrow_hbm,
                 d0_v, d1_v, w0_v, w1_v, src_v, wr_v, rows_a, rows_b,
                 sem_a, sem_b):
        wid = lax.axis_index("s") * 2 + lax.axis_index("c")
        rbase = wid * rows_w

        pltpu.sync_copy(d0_hbm, d0_v)
        pltpu.sync_copy(d1_hbm, d1_v)
        pltpu.sync_copy(w0_hbm, w0_v)
        pltpu.sync_copy(w1_hbm, w1_v)

        lanes = lax.broadcasted_iota(jnp.int32, (_L,), 0)

        zf = jnp.zeros((_L,), jnp.float32)
        def init_body(j, _):
            src_v[pl.ds(j * _L, _L)] = lax.rem(rbase + j * _L + lanes, T)
            wr_v[pl.ds(j * _L, _L)] = zf
            return 0
        lax.fori_loop(0, rows_w // _L, init_body, 0)

        def scan_body(j, _):
            toks = j * _L + lanes
            for dv, wv in ((d0_v, w0_v), (d1_v, w1_v)):
                idx = dv[pl.ds(j * _L, _L)] - rbase
                m = (idx >= 0) & (idx < rows_w)
                plsc.store_scatter(src_v, [idx], toks, mask=m)
                plsc.store_scatter(wr_v, [idx], wv[pl.ds(j * _L, _L)], mask=m)
            return 0
        lax.fori_loop(0, n_scan, scan_body, 0)

        pltpu.sync_copy(wr_v, wrow_hbm.at[pl.ds(rbase, rows_w)])
        bufs = (rows_a, rows_b)
        sems = (sem_a, sem_b)
        handles = [None] * nchunk
        handles[0] = pltpu.async_copy(
            hs_hbm.at[src_v.at[pl.ds(0, chunk)]], bufs[0], sems[0])
        for c in range(nchunk):
            if c + 1 < nchunk:
                handles[c + 1] = pltpu.async_copy(
                    hs_hbm.at[src_v.at[pl.ds((c + 1) * chunk, chunk)]],
                    bufs[(c + 1) % 2], sems[(c + 1) % 2])
            handles[c].wait()
            pltpu.sync_copy(bufs[c % 2],
                            xh_hbm.at[pl.ds(rbase + c * chunk, chunk)])

    return gather_k


def _make_combine(T, H, NP):
    tok_w = T // _NW
    chunk = tok_w // 2
    ncol = H // _L
    mesh = plsc.VectorSubcoreMesh(core_axis_name="c", subcore_axis_name="s")

    @functools.partial(
        pl.kernel,
        out_type=jax.ShapeDtypeStruct((T, H), jnp.float32),
        mesh=mesh,
        scratch_types=[
            pltpu.VMEM((tok_w,), jnp.int32),
            pltpu.VMEM((tok_w,), jnp.int32),
            pltpu.VMEM((chunk, H), jnp.float32),
            pltpu.VMEM((chunk, H), jnp.float32),
            pltpu.SemaphoreType.DMA,
        ],
        compiler_params=pltpu.CompilerParams(needs_layout_passes=False),
    )
    def combine_k(d0_hbm, d1_hbm, yw_hbm, out_hbm,
                  i0_v, i1_v, a_v, b_v, sem):
        wid = lax.axis_index("s") * 2 + lax.axis_index("c")
        tbase = wid * tok_w
        pltpu.sync_copy(d0_hbm.at[pl.ds(tbase, tok_w)], i0_v)
        pltpu.sync_copy(d1_hbm.at[pl.ds(tbase, tok_w)], i1_v)

        for c in range(2):
            pltpu.async_copy(
                yw_hbm.at[i0_v.at[pl.ds(c * chunk, chunk)]], a_v, sem).wait()
            pltpu.async_copy(
                yw_hbm.at[i1_v.at[pl.ds(c * chunk, chunk)]], b_v, sem).wait()

            def add_body(r, _):
                for cc in range(ncol):
                    s = pl.ds(cc * _L, _L)
                    a_v[r, s] = a_v[r, s] + b_v[r, s]
                return 0
            lax.fori_loop(0, chunk, add_body, 0)
            pltpu.sync_copy(a_v, out_hbm.at[pl.ds(tbase + c * chunk, chunk)])

    return combine_k


@jax.jit
def kernel(hidden_states, router_logits, up_weight, down_weight):
    T, H = hidden_states.shape
    E = up_weight.shape[0]
    I = down_weight.shape[1]
    NT = (T * _TOPK) // _TILE + E
    NP = NT * _TILE

    d0, d1, w0, w1, te = pl.pallas_call(
        _plan_body,
        out_shape=[
            jax.ShapeDtypeStruct((T, 1), jnp.int32),
            jax.ShapeDtypeStruct((T, 1), jnp.int32),
            jax.ShapeDtypeStruct((T, 1), jnp.float32),
            jax.ShapeDtypeStruct((T, 1), jnp.float32),
            jax.ShapeDtypeStruct((NT, 1), jnp.int32),
        ],
    )(router_logits)
    d0 = d0.reshape(T)
    d1 = d1.reshape(T)
    te = te.reshape(NT)

    xh, wrow = _make_gather(T, H, NP)(
        d0, d1, w0.reshape(T), w1.reshape(T), hidden_states)


    yw = pl.pallas_call(
        _gemm_body,
        grid_spec=pltpu.PrefetchScalarGridSpec(
            num_scalar_prefetch=1,
            grid=(NT,),
            in_specs=[
                pl.BlockSpec((_TILE, H), lambda t, te: (t, 0)),
                pl.BlockSpec((1, H, I), lambda t, te: (te[t], 0, 0)),
                pl.BlockSpec((1, H, I), lambda t, te: (te[t], 0, 1)),
                pl.BlockSpec((1, I, H), lambda t, te: (te[t], 0, 0)),
                pl.BlockSpec((_TILE, 1), lambda t, te: (t, 0)),
            ],
            out_specs=pl.BlockSpec((_TILE, H), lambda t, te: (t, 0)),
        ),
        out_shape=jax.ShapeDtypeStruct((NP, H), jnp.float32),
        compiler_params=pltpu.CompilerParams(
            dimension_semantics=("arbitrary",),
        ),
    )(te, xh, up_weight, up_weight, down_weight, wrow.reshape(NP, 1))

    return _make_combine(T, H, NP)(d0, d1, yw)


# speedup vs baseline: 1.9730x; 1.0011x over previous
import functools

import jax
import jax.numpy as jnp
from jax import lax
from jax.experimental import pallas as pl
from jax.experimental.pallas import tpu as pltpu
from jax.experimental.pallas import tpu_sc as plsc

_TOPK = 2
_TILE = 128
_L = 16
_NW = 32


def _plan_body(rl_ref, d0_ref, d1_ref, w0_ref, w1_ref, te_ref, tv_ref):
    logits = rl_ref[...]
    T, E = logits.shape
    NT = te_ref.shape[0]
    col = lax.broadcasted_iota(jnp.int32, (T, E), 1)
    m1 = jnp.max(logits, axis=1, keepdims=True)
    a1 = jnp.min(jnp.where(logits == m1, col, E), axis=1, keepdims=True)
    masked = jnp.where(col == a1, -jnp.inf, logits)
    m2 = jnp.max(masked, axis=1, keepdims=True)
    a2 = jnp.min(jnp.where(masked == m2, col, E), axis=1, keepdims=True)
    z = jnp.exp(m2 - m1)
    w0_ref[...] = 1.0 / (1.0 + z)
    w1_ref[...] = z / (1.0 + z)

    sel0 = col == a1
    sel1 = col == a2
    M = sel0.astype(jnp.int32) + sel1.astype(jnp.int32)
    x = M
    sh = 1
    while sh < T:
        x = jnp.concatenate(
            [jnp.zeros((sh, E), jnp.int32), x[:-sh, :]], axis=0) + x
        sh *= 2
    excl = x - M
    cnt = x[T - 1:T, :]
    padded = ((cnt + (_TILE - 1)) // _TILE) * _TILE
    r8 = lax.broadcasted_iota(jnp.int32, (E, E), 0)
    c8 = lax.broadcasted_iota(jnp.int32, (E, E), 1)
    U = (r8 < c8).astype(jnp.float32)
    base = jnp.dot(padded.astype(jnp.float32), U,
                   preferred_element_type=jnp.float32).astype(jnp.int32)
    destM = jnp.broadcast_to(base, (T, E)) + excl
    d0_ref[...] = jnp.sum(jnp.where(sel0, destM, 0), axis=1, keepdims=True)
    d1_ref[...] = jnp.sum(jnp.where(sel1, destM, 0), axis=1, keepdims=True)

    jt = lax.broadcasted_iota(jnp.int32, (NT, E), 0)
    endB = jnp.broadcast_to(base + padded, (NT, E))
    s = jnp.sum((jt * _TILE >= endB).astype(jnp.int32), axis=1, keepdims=True)
    te_ref[...] = jnp.minimum(s, E - 1)
    baseB = jnp.broadcast_to(base, (NT, E))
    realB = jnp.broadcast_to(base + cnt, (NT, E))
    vm = (jt * _TILE >= baseB) & (jt * _TILE < realB)
    tv_ref[...] = jnp.sum(vm.astype(jnp.int32), axis=1, keepdims=True)


def _gemm_body(te_ref, tv_ref, xh_ref, gw_ref, uw_ref, dw_ref, w_ref, yw_ref):
    t = pl.program_id(0)

    @pl.when(tv_ref[t] != 0)
    def _compute():
        xh = xh_ref[...].astype(jnp.bfloat16)
        g = jnp.dot(xh, gw_ref[0].astype(jnp.bfloat16),
                    preferred_element_type=jnp.float32)
        u = jnp.dot(xh, uw_ref[0].astype(jnp.bfloat16),
                    preferred_element_type=jnp.float32)
        act = (g * jax.nn.sigmoid(g)) * u
        y = jnp.dot(act.astype(jnp.bfloat16), dw_ref[0].astype(jnp.bfloat16),
                    preferred_element_type=jnp.float32)
        yw_ref[...] = y * w_ref[...]


def _make_gather(T, H, NP):
    rows_w = NP // _NW
    nchunk = 4
    chunk = rows_w // nchunk
    n_scan = T // _L
    mesh = plsc.VectorSubcoreMesh(core_axis_name="c", subcore_axis_name="s")

    @functools.partial(
        pl.kernel,
        out_type=[jax.ShapeDtypeStruct((NP, H), jnp.float32),
                  jax.ShapeDtypeStruct((NP,), jnp.float32)],
        mesh=mesh,
        scratch_types=[
            pltpu.VMEM((T,), jnp.int32),
            pltpu.VMEM((T,), jnp.int32),
            pltpu.VMEM((T,), jnp.float32),
            pltpu.VMEM((T,), jnp.float32),
            pltpu.VMEM((rows_w,), jnp.int32),
            pltpu.VMEM((rows_w,), jnp.float32),
            pltpu.VMEM((chunk, H), jnp.float32),
            pltpu.VMEM((chunk, H), jnp.float32),
            pltpu.SemaphoreType.DMA,
            pltpu.SemaphoreType.DMA,
        ],
        compiler_params=pltpu.CompilerParams(needs_layout_passes=False),
    )
    def gather_k(d0_hbm, d1_hbm, w0_hbm, w1_hbm, hs_hbm,
                 xh_hbm, wrow_hbm,
                 d0_v, d1_v, w0_v, w1_v, src_v, wr_v, rows_a, rows_b,
                 sem_a, sem_b):
        wid = lax.axis_index("s") * 2 + lax.axis_index("c")
        rbase = wid * rows_w

        pltpu.sync_copy(d0_hbm, d0_v)
        pltpu.sync_copy(d1_hbm, d1_v)
        pltpu.sync_copy(w0_hbm, w0_v)
        pltpu.sync_copy(w1_hbm, w1_v)

        lanes = lax.broadcasted_iota(jnp.int32, (_L,), 0)

        zf = jnp.zeros((_L,), jnp.float32)
        def init_body(j, _):
            src_v[pl.ds(j * _L, _L)] = lax.rem(rbase + j * _L + lanes, T)
            wr_v[pl.ds(j * _L, _L)] = zf
            return 0
        lax.fori_loop(0, rows_w // _L, init_body, 0)

        def scan_body(j, _):
            toks = j * _L + lanes
            for dv, wv in ((d0_v, w0_v), (d1_v, w1_v)):
                idx = dv[pl.ds(j * _L, _L)] - rbase
                m = (idx >= 0) & (idx < rows_w)
                plsc.store_scatter(src_v, [idx], toks, mask=m)
                plsc.store_scatter(wr_v, [idx], wv[pl.ds(j * _L, _L)], mask=m)
            return 0
        lax.fori_loop(0, n_scan, scan_body, 0)

        pltpu.sync_copy(wr_v, wrow_hbm.at[pl.ds(rbase, rows_w)])
        bufs = (rows_a, rows_b)
        sems = (sem_a, sem_b)
        handles = [None] * nchunk
        handles[0] = pltpu.async_copy(
            hs_hbm.at[src_v.at[pl.ds(0, chunk)]], bufs[0], sems[0])
        for c in range(nchunk):
            if c + 1 < nchunk:
                handles[c + 1] = pltpu.async_copy(
                    hs_hbm.at[src_v.at[pl.ds((c + 1) * chunk, chunk)]],
                    bufs[(c + 1) % 2], sems[(c + 1) % 2])
            handles[c].wait()
            pltpu.sync_copy(bufs[c % 2],
                            xh_hbm.at[pl.ds(rbase + c * chunk, chunk)])

    return gather_k


def _make_combine(T, H, NP):
    tok_w = T // _NW
    chunk = tok_w // 2
    ncol = H // _L
    mesh = plsc.VectorSubcoreMesh(core_axis_name="c", subcore_axis_name="s")

    @functools.partial(
        pl.kernel,
        out_type=jax.ShapeDtypeStruct((T, H), jnp.float32),
        mesh=mesh,
        scratch_types=[
            pltpu.VMEM((tok_w,), jnp.int32),
            pltpu.VMEM((tok_w,), jnp.int32),
            pltpu.VMEM((chunk, H), jnp.float32),
            pltpu.VMEM((chunk, H), jnp.float32),
            pltpu.SemaphoreType.DMA,
        ],
        compiler_params=pltpu.CompilerParams(needs_layout_passes=False),
    )
    def combine_k(d0_hbm, d1_hbm, yw_hbm, out_hbm,
                  i0_v, i1_v, a_v, b_v, sem):
        wid = lax.axis_index("s") * 2 + lax.axis_index("c")
        tbase = wid * tok_w
        pltpu.sync_copy(d0_hbm.at[pl.ds(tbase, tok_w)], i0_v)
        pltpu.sync_copy(d1_hbm.at[pl.ds(tbase, tok_w)], i1_v)

        for c in range(2):
            pltpu.async_copy(
                yw_hbm.at[i0_v.at[pl.ds(c * chunk, chunk)]], a_v, sem).wait()
            pltpu.async_copy(
                yw_hbm.at[i1_v.at[pl.ds(c * chunk, chunk)]], b_v, sem).wait()

            def add_body(r, _):
                for cc in range(ncol):
                    s = pl.ds(cc * _L, _L)
                    a_v[r, s] = a_v[r, s] + b_v[r, s]
                return 0
            lax.fori_loop(0, chunk, add_body, 0)
            pltpu.sync_copy(a_v, out_hbm.at[pl.ds(tbase + c * chunk, chunk)])

    return combine_k


@jax.jit
def kernel(hidden_states, router_logits, up_weight, down_weight):
    T, H = hidden_states.shape
    E = up_weight.shape[0]
    I = down_weight.shape[1]
    NT = (T * _TOPK) // _TILE + E
    NP = NT * _TILE

    d0, d1, w0, w1, te, tv = pl.pallas_call(
        _plan_body,
        out_shape=[
            jax.ShapeDtypeStruct((T, 1), jnp.int32),
            jax.ShapeDtypeStruct((T, 1), jnp.int32),
            jax.ShapeDtypeStruct((T, 1), jnp.float32),
            jax.ShapeDtypeStruct((T, 1), jnp.float32),
            jax.ShapeDtypeStruct((NT, 1), jnp.int32),
            jax.ShapeDtypeStruct((NT, 1), jnp.int32),
        ],
    )(router_logits)
    d0 = d0.reshape(T)
    d1 = d1.reshape(T)
    te = te.reshape(NT)
    tv = tv.reshape(NT)

    xh, wrow = _make_gather(T, H, NP)(
        d0, d1, w0.reshape(T), w1.reshape(T), hidden_states)


    yw = pl.pallas_call(
        _gemm_body,
        grid_spec=pltpu.PrefetchScalarGridSpec(
            num_scalar_prefetch=2,
            grid=(NT,),
            in_specs=[
                pl.BlockSpec((_TILE, H), lambda t, te, tv: (t, 0)),
                pl.BlockSpec((1, H, I), lambda t, te, tv: (te[t], 0, 0)),
                pl.BlockSpec((1, H, I), lambda t, te, tv: (te[t], 0, 1)),
                pl.BlockSpec((1, I, H), lambda t, te, tv: (te[t], 0, 0)),
                pl.BlockSpec((_TILE, 1), lambda t, te, tv: (t, 0)),
            ],
            out_specs=pl.BlockSpec((_TILE, H), lambda t, te, tv: (t, 0)),
        ),
        out_shape=jax.ShapeDtypeStruct((NP, H), jnp.float32),
        compiler_params=pltpu.CompilerParams(
            dimension_semantics=("arbitrary",),
        ),
    )(te, tv, xh, up_weight, up_weight, down_weight, wrow.reshape(NP, 1))

    return _make_combine(T, H, NP)(d0, d1, yw)


# speedup vs baseline: 1.9943x; 1.0108x over previous
import functools

import jax
import jax.numpy as jnp
from jax import lax
from jax.experimental import pallas as pl
from jax.experimental.pallas import tpu as pltpu
from jax.experimental.pallas import tpu_sc as plsc

_TOPK = 2
_TILE = 128
_L = 16
_NW = 32


def _plan_body(rl_ref, d0_ref, d1_ref, w0_ref, w1_ref, te_ref, tv_ref):
    logits = rl_ref[...]
    T, E = logits.shape
    NT = te_ref.shape[0]
    col = lax.broadcasted_iota(jnp.int32, (T, E), 1)
    m1 = jnp.max(logits, axis=1, keepdims=True)
    a1 = jnp.min(jnp.where(logits == m1, col, E), axis=1, keepdims=True)
    masked = jnp.where(col == a1, -jnp.inf, logits)
    m2 = jnp.max(masked, axis=1, keepdims=True)
    a2 = jnp.min(jnp.where(masked == m2, col, E), axis=1, keepdims=True)
    z = jnp.exp(m2 - m1)
    w0_ref[...] = 1.0 / (1.0 + z)
    w1_ref[...] = z / (1.0 + z)

    sel0 = col == a1
    sel1 = col == a2
    M = sel0.astype(jnp.int32) + sel1.astype(jnp.int32)
    x = M
    sh = 1
    while sh < T:
        x = jnp.concatenate(
            [jnp.zeros((sh, E), jnp.int32), x[:-sh, :]], axis=0) + x
        sh *= 2
    excl = x - M
    cnt = x[T - 1:T, :]
    padded = ((cnt + (_TILE - 1)) // _TILE) * _TILE
    r8 = lax.broadcasted_iota(jnp.int32, (E, E), 0)
    c8 = lax.broadcasted_iota(jnp.int32, (E, E), 1)
    U = (r8 < c8).astype(jnp.float32)
    base = jnp.dot(padded.astype(jnp.float32), U,
                   preferred_element_type=jnp.float32).astype(jnp.int32)
    destM = jnp.broadcast_to(base, (T, E)) + excl
    d0_ref[...] = jnp.sum(jnp.where(sel0, destM, 0), axis=1, keepdims=True)
    d1_ref[...] = jnp.sum(jnp.where(sel1, destM, 0), axis=1, keepdims=True)

    jt = lax.broadcasted_iota(jnp.int32, (NT, E), 0)
    endB = jnp.broadcast_to(base + padded, (NT, E))
    s = jnp.sum((jt * _TILE >= endB).astype(jnp.int32), axis=1, keepdims=True)
    te_ref[...] = jnp.minimum(s, E - 1)
    baseB = jnp.broadcast_to(base, (NT, E))
    realB = jnp.broadcast_to(base + cnt, (NT, E))
    vm = (jt * _TILE >= baseB) & (jt * _TILE < realB)
    tv_ref[...] = jnp.sum(vm.astype(jnp.int32), axis=1, keepdims=True)


def _gemm_body(te_ref, tv_ref, xh_ref, gw_ref, uw_ref, dw_ref, w_ref, yw_ref):
    t = pl.program_id(0)

    @pl.when(tv_ref[t] != 0)
    def _compute():
        xh = xh_ref[...].astype(jnp.bfloat16)
        g = jnp.dot(xh, gw_ref[0].astype(jnp.bfloat16),
                    preferred_element_type=jnp.float32)
        u = jnp.dot(xh, uw_ref[0].astype(jnp.bfloat16),
                    preferred_element_type=jnp.float32)
        act = (g * jax.nn.sigmoid(g)) * u
        y = jnp.dot(act.astype(jnp.bfloat16), dw_ref[0].astype(jnp.bfloat16),
                    preferred_element_type=jnp.float32)
        yw_ref[...] = y * w_ref[...]


def _make_gather(T, H, NP):
    rows_w = NP // _NW
    nchunk = 4
    chunk = rows_w // nchunk
    n_scan = T // _L
    mesh = plsc.VectorSubcoreMesh(core_axis_name="c", subcore_axis_name="s")

    @functools.partial(
        pl.kernel,
        out_type=[jax.ShapeDtypeStruct((NP, H), jnp.float32),
                  jax.ShapeDtypeStruct((NP,), jnp.float32)],
        mesh=mesh,
        scratch_types=[
            pltpu.VMEM((T,), jnp.int32),
            pltpu.VMEM((T,), jnp.int32),
            pltpu.VMEM((T,), jnp.float32),
            pltpu.VMEM((T,), jnp.float32),
            pltpu.VMEM((rows_w,), jnp.int32),
            pltpu.VMEM((rows_w,), jnp.float32),
            pltpu.VMEM((chunk, H), jnp.float32),
            pltpu.VMEM((chunk, H), jnp.float32),
            pltpu.SemaphoreType.DMA,
            pltpu.SemaphoreType.DMA,
        ],
        compiler_params=pltpu.CompilerParams(needs_layout_passes=False),
    )
    def gather_k(d0_hbm, d1_hbm, w0_hbm, w1_hbm, hs_hbm,
                 xh_hbm, wrow_hbm,
                 d0_v, d1_v, w0_v, w1_v, src_v, wr_v, rows_a, rows_b,
                 sem_a, sem_b):
        wid = lax.axis_index("s") * 2 + lax.axis_index("c")
        rbase = wid * rows_w

        pltpu.sync_copy(d0_hbm, d0_v)
        pltpu.sync_copy(d1_hbm, d1_v)
        pltpu.sync_copy(w0_hbm, w0_v)
        pltpu.sync_copy(w1_hbm, w1_v)

        lanes = lax.broadcasted_iota(jnp.int32, (_L,), 0)

        zf = jnp.zeros((_L,), jnp.float32)

        @plsc.parallel_loop(0, rows_w // _L, unroll=4)
        def _init(j):
            src_v[pl.ds(j * _L, _L)] = lax.rem(rbase + j * _L + lanes, T)
            wr_v[pl.ds(j * _L, _L)] = zf

        @plsc.parallel_loop(0, n_scan, unroll=4)
        def _scan(j):
            toks = j * _L + lanes
            for dv, wv in ((d0_v, w0_v), (d1_v, w1_v)):
                idx = dv[pl.ds(j * _L, _L)] - rbase
                m = (idx >= 0) & (idx < rows_w)
                plsc.store_scatter(src_v, [idx], toks, mask=m)
                plsc.store_scatter(wr_v, [idx], wv[pl.ds(j * _L, _L)], mask=m)

        pltpu.sync_copy(wr_v, wrow_hbm.at[pl.ds(rbase, rows_w)])
        bufs = (rows_a, rows_b)
        sems = (sem_a, sem_b)
        handles = [None] * nchunk
        handles[0] = pltpu.async_copy(
            hs_hbm.at[src_v.at[pl.ds(0, chunk)]], bufs[0], sems[0])
        for c in range(nchunk):
            if c + 1 < nchunk:
                handles[c + 1] = pltpu.async_copy(
                    hs_hbm.at[src_v.at[pl.ds((c + 1) * chunk, chunk)]],
                    bufs[(c + 1) % 2], sems[(c + 1) % 2])
            handles[c].wait()
            pltpu.sync_copy(bufs[c % 2],
                            xh_hbm.at[pl.ds(rbase + c * chunk, chunk)])

    return gather_k


def _make_combine(T, H, NP):
    tok_w = T // _NW
    chunk = tok_w // 2
    ncol = H // _L
    mesh = plsc.VectorSubcoreMesh(core_axis_name="c", subcore_axis_name="s")

    @functools.partial(
        pl.kernel,
        out_type=jax.ShapeDtypeStruct((T, H), jnp.float32),
        mesh=mesh,
        scratch_types=[
            pltpu.VMEM((tok_w,), jnp.int32),
            pltpu.VMEM((tok_w,), jnp.int32),
            pltpu.VMEM((chunk, H), jnp.float32),
            pltpu.VMEM((chunk, H), jnp.float32),
            pltpu.SemaphoreType.DMA,
        ],
        compiler_params=pltpu.CompilerParams(needs_layout_passes=False),
    )
    def combine_k(d0_hbm, d1_hbm, yw_hbm, out_hbm,
                  i0_v, i1_v, a_v, b_v, sem):
        wid = lax.axis_index("s") * 2 + lax.axis_index("c")
        tbase = wid * tok_w
        pltpu.sync_copy(d0_hbm.at[pl.ds(tbase, tok_w)], i0_v)
        pltpu.sync_copy(d1_hbm.at[pl.ds(tbase, tok_w)], i1_v)

        for c in range(2):
            pltpu.async_copy(
                yw_hbm.at[i0_v.at[pl.ds(c * chunk, chunk)]], a_v, sem).wait()
            pltpu.async_copy(
                yw_hbm.at[i1_v.at[pl.ds(c * chunk, chunk)]], b_v, sem).wait()

            @plsc.parallel_loop(0, chunk, unroll=2)
            def _add(r):
                for cc in range(ncol):
                    s = pl.ds(cc * _L, _L)
                    a_v[r, s] = a_v[r, s] + b_v[r, s]
            pltpu.sync_copy(a_v, out_hbm.at[pl.ds(tbase + c * chunk, chunk)])

    return combine_k


@jax.jit
def kernel(hidden_states, router_logits, up_weight, down_weight):
    T, H = hidden_states.shape
    E = up_weight.shape[0]
    I = down_weight.shape[1]
    NT = (T * _TOPK) // _TILE + E
    NP = NT * _TILE

    d0, d1, w0, w1, te, tv = pl.pallas_call(
        _plan_body,
        out_shape=[
            jax.ShapeDtypeStruct((T, 1), jnp.int32),
            jax.ShapeDtypeStruct((T, 1), jnp.int32),
            jax.ShapeDtypeStruct((T, 1), jnp.float32),
            jax.ShapeDtypeStruct((T, 1), jnp.float32),
            jax.ShapeDtypeStruct((NT, 1), jnp.int32),
            jax.ShapeDtypeStruct((NT, 1), jnp.int32),
        ],
    )(router_logits)
    d0 = d0.reshape(T)
    d1 = d1.reshape(T)
    te = te.reshape(NT)
    tv = tv.reshape(NT)

    xh, wrow = _make_gather(T, H, NP)(
        d0, d1, w0.reshape(T), w1.reshape(T), hidden_states)


    yw = pl.pallas_call(
        _gemm_body,
        grid_spec=pltpu.PrefetchScalarGridSpec(
            num_scalar_prefetch=2,
            grid=(NT,),
            in_specs=[
                pl.BlockSpec((_TILE, H), lambda t, te, tv: (t, 0)),
                pl.BlockSpec((1, H, I), lambda t, te, tv: (te[t], 0, 0)),
                pl.BlockSpec((1, H, I), lambda t, te, tv: (te[t], 0, 1)),
                pl.BlockSpec((1, I, H), lambda t, te, tv: (te[t], 0, 0)),
                pl.BlockSpec((_TILE, 1), lambda t, te, tv: (t, 0)),
            ],
            out_specs=pl.BlockSpec((_TILE, H), lambda t, te, tv: (t, 0)),
        ),
        out_shape=jax.ShapeDtypeStruct((NP, H), jnp.float32),
        compiler_params=pltpu.CompilerParams(
            dimension_semantics=("arbitrary",),
        ),
    )(te, tv, xh, up_weight, up_weight, down_weight, wrow.reshape(NP, 1))

    return _make_combine(T, H, NP)(d0, d1, yw)
